# Initial kernel scaffold; baseline (speedup 1.0000x reference)
import jax
import jax.numpy as jnp
from jax.experimental import pallas as pl


def kernel(inputs, adj, cmt_weight, W, a):
    raise NotImplementedError("write your pallas kernel here")


# speedup vs baseline: 1.6876x; 1.6876x over previous
import functools

import jax
import jax.numpy as jnp
from jax.experimental import pallas as pl

N = 4096
D = 256
ALPHA = 0.2
BLOCK = 512


def _proj_kernel(x_ref, w_ref, a_ref, wh_ref, s1_ref, s2_ref):
    wh = jnp.dot(x_ref[...], w_ref[...], preferred_element_type=jnp.float32)
    wh_ref[...] = wh
    s1_ref[...] = jnp.dot(wh, a_ref[:D, :], preferred_element_type=jnp.float32)
    s2_ref[...] = jnp.dot(wh, a_ref[D:, :], preferred_element_type=jnp.float32)


def _attn_kernel(adj_ref, s1_ref, s2t_ref, wh_ref, out_ref):
    i = pl.program_id(0)
    s1_blk = s1_ref[pl.ds(i * BLOCK, BLOCK), :]
    e = s1_blk + s2t_ref[...]
    e = jnp.where(e >= 0, e, ALPHA * e)
    att = jnp.where(adj_ref[...] > 0, e, jnp.float32(-9e15))
    m = jnp.max(att, axis=-1, keepdims=True)
    p = jnp.exp(att - m)
    denom = jnp.sum(p, axis=-1, keepdims=True)
    acc = jnp.dot(p, wh_ref[...], preferred_element_type=jnp.float32)
    out_ref[...] = jnp.maximum(acc / denom, 0.0)


@jax.jit
def kernel(inputs, adj, cmt_weight, W, a):
    del cmt_weight
    wh, s1, s2 = pl.pallas_call(
        _proj_kernel,
        out_shape=(
            jax.ShapeDtypeStruct((N, D), jnp.float32),
            jax.ShapeDtypeStruct((N, 1), jnp.float32),
            jax.ShapeDtypeStruct((N, 1), jnp.float32),
        ),
    )(inputs, W, a)

    s2t = s2.reshape(1, N)
    grid = N // BLOCK
    out = pl.pallas_call(
        _attn_kernel,
        grid=(grid,),
        in_specs=[
            pl.BlockSpec((BLOCK, N), lambda i: (i, 0)),
            pl.BlockSpec((N, 1), lambda i: (0, 0)),
            pl.BlockSpec((1, N), lambda i: (0, 0)),
            pl.BlockSpec((N, D), lambda i: (0, 0)),
        ],
        out_specs=pl.BlockSpec((BLOCK, D), lambda i: (i, 0)),
        out_shape=jax.ShapeDtypeStruct((N, D), jnp.float32),
    )(adj, s1, s2t, wh)
    return out


# speedup vs baseline: 1.7375x; 1.0296x over previous
import math

import jax
import jax.numpy as jnp
from jax.experimental import pallas as pl

N = 4096
D = 256
ALPHA = 0.2
LOG2E = math.log2(math.e)
BIGL = 9e15 * LOG2E
BLOCK = 512


def _proj_kernel(x_ref, w_ref, a_ref, whb_ref, s1_ref, s2_ref):
    wh = jnp.dot(x_ref[...], w_ref[...], preferred_element_type=jnp.float32)
    whb_ref[...] = wh.astype(jnp.bfloat16)
    s1_ref[...] = LOG2E * jnp.dot(wh, a_ref[:D, :],
                                  preferred_element_type=jnp.float32)
    s2_ref[...] = LOG2E * jnp.dot(wh, a_ref[D:, :],
                                  preferred_element_type=jnp.float32)


def _attn_kernel(adj_ref, s1_ref, s2t_ref, whb_ref, out_ref):
    i = pl.program_id(0)
    s1_blk = s1_ref[pl.ds(i * BLOCK, BLOCK), :]
    t = s1_blk + s2t_ref[...]
    u = jnp.maximum(t, ALPHA * t)
    att = jnp.where(adj_ref[...] > 0, u, jnp.float32(-BIGL))
    m = jnp.max(att, axis=-1, keepdims=True)
    p = jnp.exp2(att - m)
    denom = jnp.sum(p, axis=-1, keepdims=True)
    acc = jnp.dot(p.astype(jnp.bfloat16), whb_ref[...],
                  preferred_element_type=jnp.float32)
    out_ref[...] = jnp.maximum(acc / denom, 0.0)


@jax.jit
def kernel(inputs, adj, cmt_weight, W, a):
    del cmt_weight
    whb, s1, s2 = pl.pallas_call(
        _proj_kernel,
        out_shape=(
            jax.ShapeDtypeStruct((N, D), jnp.bfloat16),
            jax.ShapeDtypeStruct((N, 1), jnp.float32),
            jax.ShapeDtypeStruct((N, 1), jnp.float32),
        ),
    )(inputs, W, a)

    s2t = s2.reshape(1, N)
    grid = N // BLOCK
    out = pl.pallas_call(
        _attn_kernel,
        grid=(grid,),
        in_specs=[
            pl.BlockSpec((BLOCK, N), lambda i: (i, 0)),
            pl.BlockSpec((N, 1), lambda i: (0, 0)),
            pl.BlockSpec((1, N), lambda i: (0, 0)),
            pl.BlockSpec((N, D), lambda i: (0, 0)),
        ],
        out_specs=pl.BlockSpec((BLOCK, D), lambda i: (i, 0)),
        out_shape=jax.ShapeDtypeStruct((N, D), jnp.float32),
    )(adj, s1, s2t, whb)
    return out


# speedup vs baseline: 2.5631x; 1.4752x over previous
import math

import jax
import jax.numpy as jnp
from jax.experimental import pallas as pl
from jax.experimental.pallas import tpu as pltpu

N = 4096
D = 256
DE = D + 128
ALPHA = 0.2
LOG2E = math.log2(math.e)
NEG = -16384.0
BLOCK = 512


def _gat_kernel(x_ref, w_ref, a_ref, adj_ref, out_ref,
                whe_ref, s1_ref, s2t_ref, m_ref, mean_ref):
    i = pl.program_id(0)

    @pl.when(i == 0)
    def _proj():
        wh = jnp.dot(x_ref[...], w_ref[...],
                     preferred_element_type=jnp.float32)
        whe_ref[...] = jnp.concatenate(
            [wh.astype(jnp.bfloat16),
             jnp.full((N, DE - D), 1, dtype=jnp.bfloat16)], axis=1)
        s1 = LOG2E * jnp.dot(wh, a_ref[:D, :],
                             preferred_element_type=jnp.float32)
        s2t = LOG2E * jax.lax.dot_general(
            a_ref[D:, :], wh, (((0,), (1,)), ((), ())),
            preferred_element_type=jnp.float32)
        s1_ref[...] = s1
        s2t_ref[...] = s2t
        m_ref[...] = jnp.abs(s1) + jnp.max(jnp.abs(s2t))
        mean_ref[...] = jnp.sum(wh, axis=0, keepdims=True) * (1.0 / N)

    rows = pl.ds(i * BLOCK, BLOCK)
    t = s1_ref[rows, :] + s2t_ref[...]
    u = jnp.maximum(t, ALPHA * t)
    arg = jnp.where(adj_ref[...] > 0, u - m_ref[rows, :], NEG)
    p = jnp.exp2(arg).astype(jnp.bfloat16)
    res = jnp.dot(p, whe_ref[...], preferred_element_type=jnp.float32)
    acc = res[:, :D]
    denom = res[:, D:D + 1]
    h = jnp.where(denom > 0, acc / denom, mean_ref[...])
    out_ref[...] = jnp.maximum(h, 0.0)


@jax.jit
def kernel(inputs, adj, cmt_weight, W, a):
    del cmt_weight
    grid = N // BLOCK
    out = pl.pallas_call(
        _gat_kernel,
        grid=(grid,),
        in_specs=[
            pl.BlockSpec((N, D), lambda i: (0, 0)),
            pl.BlockSpec((D, D), lambda i: (0, 0)),
            pl.BlockSpec((2 * D, 1), lambda i: (0, 0)),
            pl.BlockSpec((BLOCK, N), lambda i: (i, 0)),
        ],
        out_specs=pl.BlockSpec((BLOCK, D), lambda i: (i, 0)),
        out_shape=jax.ShapeDtypeStruct((N, D), jnp.float32),
        scratch_shapes=[
            pltpu.VMEM((N, DE), jnp.bfloat16),
            pltpu.VMEM((N, 1), jnp.float32),
            pltpu.VMEM((1, N), jnp.float32),
            pltpu.VMEM((N, 1), jnp.float32),
            pltpu.VMEM((1, D), jnp.float32),
        ],
    )(inputs, W, a, adj)
    return out


# speedup vs baseline: 2.5992x; 1.0141x over previous
import math

import jax
import jax.numpy as jnp
from jax.experimental import pallas as pl
from jax.experimental.pallas import tpu as pltpu

N = 4096
D = 256
DE = D + 128
ALPHA = 0.2
LOG2E = math.log2(math.e)
NEG = -16384.0
BLOCK = 1024


def _gat_kernel(x_ref, w_ref, a_ref, adj_ref, out_ref,
                whe_ref, s1_ref, s2t_ref, m_ref, mean_ref):
    i = pl.program_id(0)

    @pl.when(i == 0)
    def _proj():
        wh = jnp.dot(x_ref[...], w_ref[...],
                     preferred_element_type=jnp.float32)
        whe_ref[...] = jnp.concatenate(
            [wh.astype(jnp.bfloat16),
             jnp.full((N, DE - D), 1, dtype=jnp.bfloat16)], axis=1)
        s1 = LOG2E * jnp.dot(wh, a_ref[:D, :],
                             preferred_element_type=jnp.float32)
        s2t = LOG2E * jax.lax.dot_general(
            a_ref[D:, :], wh, (((0,), (1,)), ((), ())),
            preferred_element_type=jnp.float32)
        s1_ref[...] = s1
        s2t_ref[...] = s2t
        m_ref[...] = jnp.abs(s1) + jnp.max(jnp.abs(s2t))
        mean_ref[...] = jnp.sum(wh, axis=0, keepdims=True) * (1.0 / N)

    rows = pl.ds(i * BLOCK, BLOCK)
    t = s1_ref[rows, :] + s2t_ref[...]
    u = jnp.maximum(t, ALPHA * t)
    arg = jnp.where(adj_ref[...] > 0, u - m_ref[rows, :], NEG)
    p = jnp.exp2(arg).astype(jnp.bfloat16)
    res = jnp.dot(p, whe_ref[...], preferred_element_type=jnp.float32)
    acc = res[:, :D]
    denom = res[:, D:D + 1]
    h = jnp.where(denom > 0, acc / denom, mean_ref[...])
    out_ref[...] = jnp.maximum(h, 0.0)


@jax.jit
def kernel(inputs, adj, cmt_weight, W, a):
    del cmt_weight
    grid = N // BLOCK
    out = pl.pallas_call(
        _gat_kernel,
        grid=(grid,),
        in_specs=[
            pl.BlockSpec((N, D), lambda i: (0, 0)),
            pl.BlockSpec((D, D), lambda i: (0, 0)),
            pl.BlockSpec((2 * D, 1), lambda i: (0, 0)),
            pl.BlockSpec((BLOCK, N), lambda i: (i, 0)),
        ],
        out_specs=pl.BlockSpec((BLOCK, D), lambda i: (i, 0)),
        out_shape=jax.ShapeDtypeStruct((N, D), jnp.float32),
        scratch_shapes=[
            pltpu.VMEM((N, DE), jnp.bfloat16),
            pltpu.VMEM((N, 1), jnp.float32),
            pltpu.VMEM((1, N), jnp.float32),
            pltpu.VMEM((N, 1), jnp.float32),
            pltpu.VMEM((1, D), jnp.float32),
        ],
    )(inputs, W, a, adj)
    return out
